# Initial kernel scaffold; baseline (speedup 1.0000x reference)
import functools

import jax
import jax.numpy as jnp
from jax import lax
from jax.experimental import pallas as pl
from jax.experimental.pallas import tpu as pltpu
from jax.experimental.pallas import tpu_sc as plsc

N = 6_400_000
NSEG = 100_000
NC = 2
NS = 16
LANES = 16
NW = NC * NS

ROW = 64
NROWS = N // ROW
ROWS_PER_W = NROWS // NW
BLK_ROWS = 125
NBLK = ROWS_PER_W // BLK_ROWS
SEG_PER_TILE = 6272
NSEG_PAD = NS * SEG_PER_TILE


def _sc_segment_sums(preds2, target2, map2, zeros_pad):
    mesh = plsc.VectorSubcoreMesh(core_axis_name="c", subcore_axis_name="s")

    @functools.partial(
        pl.kernel,
        out_type=jax.ShapeDtypeStruct((NC, NSEG_PAD), jnp.float32),
        mesh=mesh,
        scratch_types=[
            pltpu.VMEM((3, BLK_ROWS, ROW), jnp.float32),
            pltpu.VMEM((3, BLK_ROWS, ROW), jnp.float32),
            pltpu.VMEM((3, BLK_ROWS, ROW), jnp.int32),
            pltpu.VMEM_SHARED((NSEG_PAD,), jnp.float32),
            pltpu.SemaphoreType.DMA,
            pltpu.SemaphoreType.DMA,
        ],
    )
    def seg_kernel(p_hbm, t_hbm, m_hbm, z_hbm, out_hbm,
                   pbuf, tbuf, ibuf, acc, sem_in, sem_sc):
        cid = lax.axis_index("c")
        sid = lax.axis_index("s")
        wid = cid * NS + sid
        row0 = wid * ROWS_PER_W
        tile_lo = sid * SEG_PER_TILE

        pltpu.sync_copy(z_hbm.at[pl.ds(tile_lo, SEG_PER_TILE)],
                        acc.at[pl.ds(tile_lo, SEG_PER_TILE)])
        plsc.subcore_barrier()

        def fire_in(b):
            s = b % 3
            r = row0 + b * BLK_ROWS
            return (
                pltpu.async_copy(p_hbm.at[pl.ds(r, BLK_ROWS)], pbuf.at[s], sem_in),
                pltpu.async_copy(t_hbm.at[pl.ds(r, BLK_ROWS)], tbuf.at[s], sem_in),
                pltpu.async_copy(m_hbm.at[pl.ds(r, BLK_ROWS)], ibuf.at[s], sem_in),
            )

        in_handles = {0: fire_in(0)}
        sc_handles = {}
        for b in range(NBLK):
            s = b % 3
            if b >= 2:
                sc_handles.pop(b - 2).wait()
            if b + 1 < NBLK:
                in_handles[b + 1] = fire_in(b + 1)
            for h in in_handles.pop(b):
                h.wait()

            def body(r, carry, s=s):
                for c in range(ROW // LANES):
                    sl = pl.ds(c * LANES, LANES)
                    d = pbuf[s, r, sl] - tbuf[s, r, sl]
                    pbuf[s, r, sl] = d * d
                return carry

            lax.fori_loop(0, BLK_ROWS, body, None)
            sc_handles[b] = pltpu.async_copy(
                pbuf.at[s], acc.at[ibuf.at[s]], sem_sc, add=True)

        sc_handles.pop(NBLK - 2).wait()
        sc_handles.pop(NBLK - 1).wait()
        plsc.subcore_barrier()

        pltpu.sync_copy(acc.at[pl.ds(tile_lo, SEG_PER_TILE)],
                        out_hbm.at[cid, pl.ds(tile_lo, SEG_PER_TILE)])

    return seg_kernel(preds2, target2, map2, zeros_pad)


def _finalize_kernel(x_ref, o_ref):
    x = x_ref[...]
    total = x[0:1, :] + x[1:2, :]
    o_ref[0, 0] = jnp.sum(jnp.sqrt(total)) * (1.0 / NSEG)


def _finalize(partials):
    out = pl.pallas_call(
        _finalize_kernel,
        out_shape=jax.ShapeDtypeStruct((1, 1), jnp.float32),
    )(partials)
    return out[0, 0]


def kernel(preds, target, batch_map):
    preds2 = preds.reshape(NROWS, ROW)
    target2 = target.reshape(NROWS, ROW)
    map2 = batch_map.astype(jnp.int32).reshape(NROWS, ROW)
    zeros_pad = jnp.zeros((NSEG_PAD,), jnp.float32)
    partials = _sc_segment_sums(preds2, target2, map2, zeros_pad)
    return _finalize(partials)


# speedup vs baseline: 29.4910x; 29.4910x over previous
import functools

import jax
import jax.numpy as jnp
from jax import lax
from jax.experimental import pallas as pl
from jax.experimental.pallas import tpu as pltpu
from jax.experimental.pallas import tpu_sc as plsc

N = 6_400_000
NSEG = 100_000
NC = 2
NS = 16
LANES = 16
NW = NC * NS

ELEMS_PER_W = N // NW
BLK = 8_000
NBLK = ELEMS_PER_W // BLK
SEG_PER_TILE = 6272
NSEG_PAD = NS * SEG_PER_TILE


def _sc_segment_sums(preds, target, seg_ids, zeros_pad):
    mesh = plsc.VectorSubcoreMesh(core_axis_name="c", subcore_axis_name="s")

    @functools.partial(
        pl.kernel,
        out_type=jax.ShapeDtypeStruct((NC, NSEG_PAD), jnp.float32),
        mesh=mesh,
        scratch_types=[
            pltpu.VMEM((BLK,), jnp.float32),
            pltpu.VMEM((BLK,), jnp.float32),
            pltpu.VMEM((BLK,), jnp.float32),
            pltpu.VMEM((BLK,), jnp.float32),
            pltpu.VMEM((BLK,), jnp.float32),
            pltpu.VMEM((BLK,), jnp.float32),
            pltpu.VMEM((BLK,), jnp.int32),
            pltpu.VMEM((BLK,), jnp.int32),
            pltpu.VMEM((BLK,), jnp.int32),
            pltpu.VMEM_SHARED((NSEG_PAD,), jnp.float32),
            pltpu.SemaphoreType.DMA,
            pltpu.SemaphoreType.DMA,
        ],
    )
    def seg_kernel(p_hbm, t_hbm, m_hbm, z_hbm, out_hbm,
                   pb0, pb1, pb2, tb0, tb1, tb2, ib0, ib1, ib2,
                   acc, sem_in, sem_sc):
        pbufs, tbufs, ibufs = (pb0, pb1, pb2), (tb0, tb1, tb2), (ib0, ib1, ib2)
        cid = lax.axis_index("c")
        sid = lax.axis_index("s")
        wid = cid * NS + sid
        elem0 = wid * ELEMS_PER_W
        tile_lo = sid * SEG_PER_TILE

        pltpu.sync_copy(z_hbm.at[pl.ds(tile_lo, SEG_PER_TILE)],
                        acc.at[pl.ds(tile_lo, SEG_PER_TILE)])
        plsc.subcore_barrier()

        def fire_in(b):
            s = b % 3
            e = elem0 + b * BLK
            return (
                pltpu.async_copy(p_hbm.at[pl.ds(e, BLK)], pbufs[s], sem_in),
                pltpu.async_copy(t_hbm.at[pl.ds(e, BLK)], tbufs[s], sem_in),
                pltpu.async_copy(m_hbm.at[pl.ds(e, BLK)], ibufs[s], sem_in),
            )

        in_handles = {0: fire_in(0)}
        sc_handles = {}
        for b in range(NBLK):
            s = b % 3
            if b >= 2:
                sc_handles.pop(b - 2).wait()
            if b + 1 < NBLK:
                in_handles[b + 1] = fire_in(b + 1)
            for h in in_handles.pop(b):
                h.wait()

            pbuf, tbuf = pbufs[s], tbufs[s]

            def body(r, carry, pbuf=pbuf, tbuf=tbuf):
                sl = pl.ds(r * LANES, LANES)
                d = pbuf[sl] - tbuf[sl]
                pbuf[sl] = d * d
                return carry

            lax.fori_loop(0, BLK // LANES, body, None)
            sc_handles[b] = pltpu.async_copy(
                pbufs[s], acc.at[ibufs[s]], sem_sc, add=True)

        sc_handles.pop(NBLK - 2).wait()
        sc_handles.pop(NBLK - 1).wait()
        plsc.subcore_barrier()

        pltpu.sync_copy(acc.at[pl.ds(tile_lo, SEG_PER_TILE)],
                        out_hbm.at[cid, pl.ds(tile_lo, SEG_PER_TILE)])

    return seg_kernel(preds, target, seg_ids, zeros_pad)


def _finalize_kernel(x_ref, o_ref):
    x = x_ref[...]
    total = x[0:1, :] + x[1:2, :]
    o_ref[...] = jnp.reshape(jnp.sum(jnp.sqrt(total)) * (1.0 / NSEG), (1, 1))


def _finalize(partials):
    out = pl.pallas_call(
        _finalize_kernel,
        out_shape=jax.ShapeDtypeStruct((1, 1), jnp.float32),
    )(partials)
    return out[0, 0]


def kernel(preds, target, batch_map):
    seg_ids = batch_map.astype(jnp.int32)
    zeros_pad = jnp.zeros((NSEG_PAD,), jnp.float32)
    partials = _sc_segment_sums(preds, target, seg_ids, zeros_pad)
    return _finalize(partials)
